# baseline (speedup 1.0000x reference)
import functools

import jax
import jax.numpy as jnp
from jax import lax
from jax.experimental import pallas as pl
from jax.experimental.pallas import tpu as pltpu
from jax.experimental.pallas import tpu_sc as plsc

_VOCAB = 1000000
_HIDDEN = 64
_EMBED = 64
_B = 16384
_L = 200

_ROWS_PER_BLOCK = 32
_U = 4096
_NBLK = -(-_VOCAB // (2 * _U))


def _tc_pack_transpose(tableT):
    def body(x_ref, o_ref):
        x = x_ref[...]
        o_ref[:, 0:_HIDDEN] = x[:, : _U].T
        o_ref[:, _HIDDEN:] = x[:, _U:].T

    return pl.pallas_call(
        body,
        grid=(_NBLK,),
        in_specs=[pl.BlockSpec((_HIDDEN, 2 * _U), lambda i: (0, i))],
        out_specs=pl.BlockSpec((_U, 2 * _HIDDEN), lambda i: (i, 0)),
        out_shape=jax.ShapeDtypeStruct((_NBLK * _U, 2 * _HIDDEN),
                                       jnp.float32),
    )(tableT)


def _sc_pool(tokens2d, table_lin):
    info = plsc.get_sparse_core_info()
    nc, ns = info.num_cores, info.num_subcores
    nw = nc * ns
    rows_per_w = _B // nw
    npairs = rows_per_w // 2
    idx_per_block = _ROWS_PER_BLOCK * _L
    nchunks = idx_per_block // 16
    inv_l = jnp.float32(1.0 / _L)

    mesh = plsc.VectorSubcoreMesh(core_axis_name="c", subcore_axis_name="s")

    @functools.partial(
        pl.kernel,
        mesh=mesh,
        out_type=jax.ShapeDtypeStruct((_B, _HIDDEN), jnp.float32),
        scratch_types=[
            pltpu.VMEM((2, _ROWS_PER_BLOCK, _L), jnp.int32),
            pltpu.VMEM((_L, _HIDDEN), jnp.float32),
            pltpu.VMEM((_L, _HIDDEN), jnp.float32),
            pltpu.VMEM((_L, _HIDDEN), jnp.float32),
            pltpu.VMEM((_L, _HIDDEN), jnp.float32),
            pltpu.VMEM((_ROWS_PER_BLOCK, _HIDDEN), jnp.float32),
            pltpu.SemaphoreType.DMA,
            pltpu.SemaphoreType.DMA,
            pltpu.SemaphoreType.DMA,
            pltpu.SemaphoreType.DMA,
        ],
        compiler_params=pltpu.CompilerParams(use_tc_tiling_on_sc=False),
    )
    def pool(tokens_hbm, table_hbm, out_hbm, idx_v, buf0, buf1, buf2, buf3,
             pooled_v, sem0, sem1, sem2, sem3):
        wid = lax.axis_index("s") * nc + lax.axis_index("c")
        row0 = wid * rows_per_w

        def stage(blk):
            pltpu.sync_copy(
                tokens_hbm.at[pl.ds(row0 + blk * _ROWS_PER_BLOCK,
                                    _ROWS_PER_BLOCK)],
                idx_v.at[blk % 2])

        def _desc(r, buf, sem):
            blk = r // _ROWS_PER_BLOCK
            j = r % _ROWS_PER_BLOCK
            return pltpu.make_async_copy(
                table_hbm.at[idx_v.at[blk % 2, j]], buf, sem)

        def fire(r, buf, sem):
            _desc(r, buf, sem).start()

        def drain_acc(r, buf, sem):
            _desc(r, buf, sem).wait()

            def acc_body(l, accs):
                a0, a1, a2, a3 = accs
                a0 = a0 + buf[l, pl.ds(0, 16)]
                a1 = a1 + buf[l, pl.ds(16, 16)]
                a2 = a2 + buf[l, pl.ds(32, 16)]
                a3 = a3 + buf[l, pl.ds(48, 16)]
                return (a0, a1, a2, a3)

            z = jnp.zeros((16,), jnp.float32)
            a0, a1, a2, a3 = lax.fori_loop(0, _L, acc_body, (z, z, z, z),
                                           unroll=10)
            j = r % _ROWS_PER_BLOCK
            pooled_v[j, pl.ds(0, 16)] = a0 * inv_l
            pooled_v[j, pl.ds(16, 16)] = a1 * inv_l
            pooled_v[j, pl.ds(32, 16)] = a2 * inv_l
            pooled_v[j, pl.ds(48, 16)] = a3 * inv_l

        bufs = (buf0, buf1)
        sems = (sem0, sem1)

        stage(0)
        fire(0, buf0, sem0)

        def pair_body(i, carry):
            r = 2 * i
            fire(r + 1, buf1, sem1)

            for k in range(2):
                drain_acc(r + k, bufs[k], sems[k])

                @pl.when(jnp.logical_and(
                    (r + k + 2) % _ROWS_PER_BLOCK == 0,
                    r + k + 2 < rows_per_w))
                def _(k=k):
                    stage((r + k + 2) // _ROWS_PER_BLOCK)

                @pl.when(jnp.logical_and(k == 0, r + k + 2 < rows_per_w))
                def _(k=k):
                    fire(r + k + 2, bufs[k], sems[k])

            @pl.when((r + 1) % _ROWS_PER_BLOCK == _ROWS_PER_BLOCK - 1)
            def _():
                blk = (r + 1) // _ROWS_PER_BLOCK
                pltpu.sync_copy(
                    pooled_v,
                    out_hbm.at[pl.ds(row0 + blk * _ROWS_PER_BLOCK,
                                     _ROWS_PER_BLOCK)])

            return carry

        lax.fori_loop(0, rows_per_w // 2, pair_body, 0)

    return pool(tokens2d, table_lin)


def _tc_proj(pooled, W, b2):
    blk = 512

    def body(x_ref, w_ref, b_ref, o_ref):
        x = x_ref[...]
        w = w_ref[...]
        y = lax.dot_general(x, w, (((1,), (1,)), ((), ())),
                            preferred_element_type=jnp.float32)
        y = y + b_ref[...]
        n = jnp.sqrt(jnp.sum(y * y, axis=-1, keepdims=True))
        o_ref[...] = y / jnp.maximum(n, 1e-12)

    return pl.pallas_call(
        body,
        grid=(_B // blk,),
        in_specs=[
            pl.BlockSpec((blk, _HIDDEN), lambda i: (i, 0)),
            pl.BlockSpec((_EMBED, _HIDDEN), lambda i: (0, 0)),
            pl.BlockSpec((1, _EMBED), lambda i: (0, 0)),
        ],
        out_specs=pl.BlockSpec((blk, _EMBED), lambda i: (i, 0)),
        out_shape=jax.ShapeDtypeStruct((_B, _EMBED), jnp.float32),
    )(pooled, W, b2)


def kernel(token_ids, table, W, b):
    tableT = jnp.swapaxes(table, 0, 1)
    packed = _tc_pack_transpose(tableT)
    table_lin = packed.reshape(2 * _NBLK * _U, _HIDDEN)
    v = token_ids.astype(jnp.int32)
    q = jnp.bitwise_and(v, 2 * _U - 1)
    two = q + q
    remapped = v - q + jnp.where(q < _U, two, two - (2 * _U - 1))
    pooled = _sc_pool(remapped, table_lin)
    return _tc_proj(pooled, W, b.reshape(1, _EMBED))


# speedup vs baseline: 1.2551x; 1.2551x over previous
import functools

import jax
import jax.numpy as jnp
from jax import lax
from jax.experimental import pallas as pl
from jax.experimental.pallas import tpu as pltpu
from jax.experimental.pallas import tpu_sc as plsc

_VOCAB = 1000000
_HIDDEN = 64
_EMBED = 64
_B = 16384
_L = 200

_ROWS_PER_BLOCK = 32
_U = 4096
_NBLK = -(-_VOCAB // (2 * _U))


def _tc_pack_transpose(tableT):
    def body(x_ref, o_ref):
        x = x_ref[...]
        o_ref[:, 0:_HIDDEN] = x[:, : _U].T
        o_ref[:, _HIDDEN:] = x[:, _U:].T

    return pl.pallas_call(
        body,
        grid=(_NBLK,),
        in_specs=[pl.BlockSpec((_HIDDEN, 2 * _U), lambda i: (0, i))],
        out_specs=pl.BlockSpec((_U, 2 * _HIDDEN), lambda i: (i, 0)),
        out_shape=jax.ShapeDtypeStruct((_NBLK * _U, 2 * _HIDDEN),
                                       jnp.float32),
    )(tableT)


def _sc_pool(tokens2d, table_lin):
    info = plsc.get_sparse_core_info()
    nc, ns = info.num_cores, info.num_subcores
    nw = nc * ns
    rows_per_w = _B // nw
    npairs = rows_per_w // 2
    idx_per_block = _ROWS_PER_BLOCK * _L
    nchunks = idx_per_block // 16
    inv_l = jnp.float32(1.0 / _L)

    mesh = plsc.VectorSubcoreMesh(core_axis_name="c", subcore_axis_name="s")

    @functools.partial(
        pl.kernel,
        mesh=mesh,
        out_type=jax.ShapeDtypeStruct((_B, _HIDDEN), jnp.float32),
        scratch_types=[
            pltpu.VMEM((2, _ROWS_PER_BLOCK, _L), jnp.int32),
            pltpu.VMEM((_L, _HIDDEN), jnp.float32),
            pltpu.VMEM((_L, _HIDDEN), jnp.float32),
            pltpu.VMEM((_L, _HIDDEN), jnp.float32),
            pltpu.VMEM((_L, _HIDDEN), jnp.float32),
            pltpu.VMEM((_ROWS_PER_BLOCK, _HIDDEN), jnp.float32),
            pltpu.SemaphoreType.DMA,
            pltpu.SemaphoreType.DMA,
            pltpu.SemaphoreType.DMA,
            pltpu.SemaphoreType.DMA,
        ],
        compiler_params=pltpu.CompilerParams(use_tc_tiling_on_sc=False),
    )
    def pool(tokens_hbm, table_hbm, out_hbm, idx_v, buf0, buf1, buf2, buf3,
             pooled_v, sem0, sem1, sem2, sem3):
        wid = lax.axis_index("s") * nc + lax.axis_index("c")
        row0 = wid * rows_per_w

        def stage(blk):
            pltpu.sync_copy(
                tokens_hbm.at[pl.ds(row0 + blk * _ROWS_PER_BLOCK,
                                    _ROWS_PER_BLOCK)],
                idx_v.at[blk % 2])

        def _desc(r, buf, sem):
            blk = r // _ROWS_PER_BLOCK
            j = r % _ROWS_PER_BLOCK
            return pltpu.make_async_copy(
                table_hbm.at[idx_v.at[blk % 2, j]], buf, sem)

        def fire(r, buf, sem):
            _desc(r, buf, sem).start()

        def drain_acc(r, buf, sem):
            _desc(r, buf, sem).wait()

            def acc_body(l, accs):
                a0, a1, a2, a3 = accs
                a0 = a0 + buf[l, pl.ds(0, 16)]
                a1 = a1 + buf[l, pl.ds(16, 16)]
                a2 = a2 + buf[l, pl.ds(32, 16)]
                a3 = a3 + buf[l, pl.ds(48, 16)]
                return (a0, a1, a2, a3)

            z = jnp.zeros((16,), jnp.float32)
            a0, a1, a2, a3 = lax.fori_loop(0, _L, acc_body, (z, z, z, z),
                                           unroll=10)
            j = r % _ROWS_PER_BLOCK
            pooled_v[j, pl.ds(0, 16)] = a0 * inv_l
            pooled_v[j, pl.ds(16, 16)] = a1 * inv_l
            pooled_v[j, pl.ds(32, 16)] = a2 * inv_l
            pooled_v[j, pl.ds(48, 16)] = a3 * inv_l

        bufs = (buf0, buf1, buf2, buf3)
        sems = (sem0, sem1, sem2, sem3)

        stage(0)
        fire(0, buf0, sem0)
        fire(1, buf1, sem1)
        fire(2, buf2, sem2)

        def quad_body(i, carry):
            r = 4 * i
            fire(r + 3, buf3, sem3)

            for k in range(4):
                drain_acc(r + k, bufs[k], sems[k])

                @pl.when(jnp.logical_and(
                    (r + k + 4) % _ROWS_PER_BLOCK == 0,
                    r + k + 4 < rows_per_w))
                def _(k=k):
                    stage((r + k + 4) // _ROWS_PER_BLOCK)

                if k < 3:
                    @pl.when(r + k + 4 < rows_per_w)
                    def _(k=k):
                        fire(r + k + 4, bufs[k], sems[k])

            @pl.when((r + 3) % _ROWS_PER_BLOCK == _ROWS_PER_BLOCK - 1)
            def _():
                blk = (r + 3) // _ROWS_PER_BLOCK
                pltpu.sync_copy(
                    pooled_v,
                    out_hbm.at[pl.ds(row0 + blk * _ROWS_PER_BLOCK,
                                     _ROWS_PER_BLOCK)])

            return carry

        lax.fori_loop(0, rows_per_w // 4, quad_body, 0)

    return pool(tokens2d, table_lin)


def _tc_proj(pooled, W, b2):
    blk = 512

    def body(x_ref, w_ref, b_ref, o_ref):
        x = x_ref[...]
        w = w_ref[...]
        y = lax.dot_general(x, w, (((1,), (1,)), ((), ())),
                            preferred_element_type=jnp.float32)
        y = y + b_ref[...]
        n = jnp.sqrt(jnp.sum(y * y, axis=-1, keepdims=True))
        o_ref[...] = y / jnp.maximum(n, 1e-12)

    return pl.pallas_call(
        body,
        grid=(_B // blk,),
        in_specs=[
            pl.BlockSpec((blk, _HIDDEN), lambda i: (i, 0)),
            pl.BlockSpec((_EMBED, _HIDDEN), lambda i: (0, 0)),
            pl.BlockSpec((1, _EMBED), lambda i: (0, 0)),
        ],
        out_specs=pl.BlockSpec((blk, _EMBED), lambda i: (i, 0)),
        out_shape=jax.ShapeDtypeStruct((_B, _EMBED), jnp.float32),
    )(pooled, W, b2)


def kernel(token_ids, table, W, b):
    tableT = jnp.swapaxes(table, 0, 1)
    packed = _tc_pack_transpose(tableT)
    table_lin = packed.reshape(2 * _NBLK * _U, _HIDDEN)
    v = token_ids.astype(jnp.int32)
    q = jnp.bitwise_and(v, 2 * _U - 1)
    two = q + q
    remapped = v - q + jnp.where(q < _U, two, two - (2 * _U - 1))
    pooled = _sc_pool(remapped, table_lin)
    return _tc_proj(pooled, W, b.reshape(1, _EMBED))
